# Initial kernel scaffold; baseline (speedup 1.0000x reference)
import functools

import jax
import jax.numpy as jnp
from jax import lax
from jax.experimental import pallas as pl
from jax.experimental.pallas import tpu as pltpu
from jax.experimental.pallas import tpu_sc as plsc

H = 128
NC = 2
NS = 16
CUT = 8.0


def _silu(x):
    return x * jax.nn.sigmoid(x)


def _sc_mesh():
    return plsc.VectorSubcoreMesh(
        core_axis_name="c", subcore_axis_name="s", num_cores=NC, num_subcores=NS
    )


def _sc_gather(table, idx):
    E = idx.shape[0]
    NW = NC * NS
    per_w = E // NW
    CH = 80
    n_ch = per_w // CH

    @functools.partial(
        pl.kernel,
        out_type=jax.ShapeDtypeStruct((E, H), jnp.float32),
        mesh=_sc_mesh(),
        scratch_types=[
            pltpu.VMEM((CH,), jnp.int32),
            pltpu.VMEM((CH, H), jnp.float32),
            pltpu.SemaphoreType.DMA,
        ],
    )
    def k(table_hbm, idx_hbm, out_hbm, idx_v, rows_v, sem):
        wid = lax.axis_index("s") * NC + lax.axis_index("c")
        base = wid * per_w

        @pl.loop(0, n_ch)
        def _(i):
            off = base + i * CH
            pltpu.sync_copy(idx_hbm.at[pl.ds(off, CH)], idx_v)
            pltpu.async_copy(table_hbm.at[idx_v], rows_v, sem).wait()
            pltpu.sync_copy(rows_v, out_hbm.at[pl.ds(off, CH)])

    return k(table, idx)


def _sc_segsum(payload, dst, n_nodes):
    E = dst.shape[0]
    per_s = E // NS
    CH = 80
    n_ch = per_s // CH
    rows_per_s = n_nodes // NS

    zeros = jnp.zeros((n_nodes, H), jnp.float32)

    @functools.partial(
        pl.kernel,
        out_type=jax.ShapeDtypeStruct((NC, n_nodes, H), jnp.float32),
        mesh=_sc_mesh(),
        scratch_types=[
            pltpu.VMEM((CH,), jnp.int32),
            pltpu.VMEM((CH, H), jnp.float32),
            pltpu.VMEM_SHARED((n_nodes, H), jnp.float32),
            pltpu.SemaphoreType.DMA,
        ],
    )
    def k(pay_hbm, dst_hbm, zero_hbm, out_hbm, idx_v, rows_v, acc_sh, sem):
        c = lax.axis_index("c")
        s = lax.axis_index("s")
        r0 = s * rows_per_s
        pltpu.sync_copy(
            zero_hbm.at[pl.ds(r0, rows_per_s)], acc_sh.at[pl.ds(r0, rows_per_s)]
        )
        plsc.subcore_barrier()

        @pl.loop(0, n_ch)
        def _(i):
            off = s * per_s + i * CH
            pltpu.sync_copy(dst_hbm.at[pl.ds(off, CH)], idx_v)
            pltpu.sync_copy(pay_hbm.at[c, pl.ds(off, CH)], rows_v)
            pltpu.sync_copy(rows_v, acc_sh.at[idx_v], add=True)

        plsc.subcore_barrier()
        pltpu.sync_copy(
            acc_sh.at[pl.ds(r0, rows_per_s)], out_hbm.at[c, pl.ds(r0, rows_per_s)]
        )

    return k(payload, dst, zeros)


def _tc_node_proj(x_s, w, b):
    n = x_s.shape[0]
    BN = 2000

    def body(x_ref, w_ref, b_ref, o_ref):
        o_ref[...] = (
            jnp.dot(x_ref[...], w_ref[...], preferred_element_type=jnp.float32)
            + b_ref[...]
        )

    return pl.pallas_call(
        body,
        grid=(n // BN,),
        in_specs=[
            pl.BlockSpec((BN, H), lambda i: (i, 0)),
            pl.BlockSpec((H, H), lambda i: (0, 0)),
            pl.BlockSpec((1, H), lambda i: (0, 0)),
        ],
        out_specs=pl.BlockSpec((BN, H), lambda i: (i, 0)),
        out_shape=jax.ShapeDtypeStruct((n, H), jnp.float32),
    )(x_s, w, b.reshape(1, H))


def _tc_edge(edge_attr, sphe_emb, torsion_emb, dist, v_g, p):
    E = edge_attr.shape[0]
    BE = 2000
    sphe_d = sphe_emb.shape[1]
    tors_d = torsion_emb.shape[1]
    mid = p["sphe_w1"].shape[1]

    def body(ea_ref, sp_ref, to_ref, di_ref, vg_ref,
             sw1_ref, sw2_ref, tw1_ref, tw2_ref,
             dvw_ref, dvb_ref, smw_ref, smb_ref,
             cma_ref, cmb_ref,
             fpw_ref, fpb_ref, cfa_ref, cfb_ref, cfbias_ref,
             fji_ref, pay_ref):
        dot = functools.partial(jnp.dot, preferred_element_type=jnp.float32)
        ea = ea_ref[...]
        d1 = dot(dot(sp_ref[...], sw1_ref[...]), sw2_ref[...])
        d2 = dot(dot(to_ref[...], tw1_ref[...]), tw2_ref[...])
        r = di_ref[...]
        cut = 0.5 * (jnp.cos(jnp.pi * r / CUT) + 1.0) * (r < CUT).astype(r.dtype)
        dv = _silu(dot(ea, dvw_ref[...]) + dvb_ref[...]) * cut
        v_j = vg_ref[...] * dv
        s = _silu(dot(v_j, smw_ref[...]) + smb_ref[...])
        pay_d = dot(s[:, :H] * d1, cma_ref[...]) + dot(s[:, H:] * d2, cmb_ref[...])
        f = _silu(dot(ea, fpw_ref[...]) + fpb_ref[...])
        f_ji = f[:, :H] + _silu(
            dot(f[:, H:2 * H] * d1, cfa_ref[...])
            + dot(f[:, 2 * H:] * d2, cfb_ref[...])
            + cfbias_ref[...]
        )
        fji_ref[...] = f_ji
        pay_ref[0] = v_j
        pay_ref[1] = pay_d

    def full(shape):
        return pl.BlockSpec(shape, lambda *_: tuple(0 for _ in shape))

    out = pl.pallas_call(
        body,
        grid=(E // BE,),
        in_specs=[
            pl.BlockSpec((BE, H), lambda i: (i, 0)),
            pl.BlockSpec((BE, sphe_d), lambda i: (i, 0)),
            pl.BlockSpec((BE, tors_d), lambda i: (i, 0)),
            pl.BlockSpec((BE, 1), lambda i: (i, 0)),
            pl.BlockSpec((BE, H), lambda i: (i, 0)),
            full((sphe_d, mid)), full((mid, H)),
            full((tors_d, mid)), full((mid, H)),
            full((H, H)), full((1, H)),
            full((H, 2 * H)), full((1, 2 * H)),
            full((H, H)), full((H, H)),
            full((H, 3 * H)), full((1, 3 * H)),
            full((H, H)), full((H, H)), full((1, H)),
        ],
        out_specs=[
            pl.BlockSpec((BE, H), lambda i: (i, 0)),
            pl.BlockSpec((2, BE, H), lambda i: (0, i, 0)),
        ],
        out_shape=[
            jax.ShapeDtypeStruct((E, H), jnp.float32),
            jax.ShapeDtypeStruct((2, E, H), jnp.float32),
        ],
    )(
        edge_attr, sphe_emb, torsion_emb, dist.reshape(E, 1), v_g,
        p["sphe_w1"], p["sphe_w2"], p["tors_w1"], p["tors_w2"],
        p["dv_w"], p["dv_b"].reshape(1, H),
        p["s_msg_w"], p["s_msg_b"].reshape(1, 2 * H),
        p["cat_msg_w"][:H], p["cat_msg_w"][H:],
        p["f_proj_w"], p["f_proj_b"].reshape(1, 3 * H),
        p["cat_f_w"][:H], p["cat_f_w"][H:], p["cat_f_b"].reshape(1, H),
    )
    return out[0], out[1]


def _tc_node_update(x_t, agg, p):
    n = x_t.shape[0]
    BN = 2000

    def body(xt_ref, agg_ref, omw_ref, omb_ref, fw_ref, fb_ref, o_ref):
        dot = functools.partial(jnp.dot, preferred_element_type=jnp.float32)
        s_msg = agg_ref[0]
        d_msg = agg_ref[1]
        om = dot(s_msg, omw_ref[...]) + omb_ref[...]
        pre = xt_ref[...] + om[:, :H] + om[:, H:] * d_msg
        o_ref[...] = _silu(dot(pre, fw_ref[...]) + fb_ref[...])

    return pl.pallas_call(
        body,
        grid=(n // BN,),
        in_specs=[
            pl.BlockSpec((BN, H), lambda i: (i, 0)),
            pl.BlockSpec((2, BN, H), lambda i: (0, i, 0)),
            pl.BlockSpec((H, 2 * H), lambda i: (0, 0)),
            pl.BlockSpec((1, 2 * H), lambda i: (0, 0)),
            pl.BlockSpec((H, H), lambda i: (0, 0)),
            pl.BlockSpec((1, H), lambda i: (0, 0)),
        ],
        out_specs=pl.BlockSpec((BN, H), lambda i: (i, 0)),
        out_shape=jax.ShapeDtypeStruct((n, H), jnp.float32),
    )(x_t, agg, p["o_msg_w"], p["o_msg_b"].reshape(1, 2 * H),
      p["final_w"], p["final_b"].reshape(1, H))


def kernel(x_s, x_t, edge_index, edge_attr, sphe_emb, torsion_emb, dist, params):
    n = x_s.shape[0]
    src = edge_index[0].astype(jnp.int32)
    dst = edge_index[1].astype(jnp.int32)
    v = _tc_node_proj(x_s, params["node_w"], params["node_b"])
    v_g = _sc_gather(v, src)
    f_ji, pay = _tc_edge(edge_attr, sphe_emb, torsion_emb, dist, v_g, params)
    agg = _sc_segsum(pay, dst, n)
    h_t = _tc_node_update(x_t, agg, params)
    return (h_t, f_ji)


# speedup vs baseline: 2.1261x; 2.1261x over previous
import functools

import jax
import jax.numpy as jnp
from jax import lax
from jax.experimental import pallas as pl
from jax.experimental.pallas import tpu as pltpu
from jax.experimental.pallas import tpu_sc as plsc

H = 128
NC = 2
NS = 16
CUT = 8.0


def _silu(x):
    return x * jax.nn.sigmoid(x)


def _sc_mesh():
    return plsc.VectorSubcoreMesh(
        core_axis_name="c", subcore_axis_name="s", num_cores=NC, num_subcores=NS
    )


def _sc_gather(table, idx):
    E = idx.shape[0]
    NW = NC * NS
    per_w = E // NW
    CH = 80
    n_ch = per_w // CH

    @functools.partial(
        pl.kernel,
        out_type=jax.ShapeDtypeStruct((E, H), jnp.float32),
        mesh=_sc_mesh(),
        scratch_types=[
            pltpu.VMEM((CH,), jnp.int32),
            pltpu.VMEM((CH, H), jnp.float32),
            pltpu.SemaphoreType.DMA,
        ],
    )
    def k(table_hbm, idx_hbm, out_hbm, idx_v, rows_v, sem):
        wid = lax.axis_index("s") * NC + lax.axis_index("c")
        base = wid * per_w

        @pl.loop(0, n_ch)
        def _(i):
            off = base + i * CH
            pltpu.sync_copy(idx_hbm.at[pl.ds(off, CH)], idx_v)
            pltpu.async_copy(table_hbm.at[idx_v], rows_v, sem).wait()
            pltpu.sync_copy(rows_v, out_hbm.at[pl.ds(off, CH)])

    return k(table, idx)


def _sc_segsum(payload, dst, n_nodes):
    E = dst.shape[0]
    per_s = E // NS
    CH = 80
    n_ch = per_s // CH
    rows_per_s = (n_nodes // NS) // 8 * 8
    tail_rows = n_nodes - (NS - 1) * rows_per_s - rows_per_s
    tail_off = NS * rows_per_s

    zeros = jnp.zeros((n_nodes, H), jnp.float32)

    @functools.partial(
        pl.kernel,
        out_type=jax.ShapeDtypeStruct((NC, n_nodes, H), jnp.float32),
        mesh=_sc_mesh(),
        scratch_types=[
            pltpu.VMEM((CH,), jnp.int32),
            pltpu.VMEM((CH, H), jnp.float32),
            pltpu.VMEM_SHARED((n_nodes, H), jnp.float32),
            pltpu.SemaphoreType.DMA,
        ],
    )
    def k(pay_hbm, dst_hbm, zero_hbm, out_hbm, idx_v, rows_v, acc_sh, sem):
        c = lax.axis_index("c")
        s = lax.axis_index("s")
        r0 = s * rows_per_s
        pltpu.sync_copy(
            zero_hbm.at[pl.ds(r0, rows_per_s)], acc_sh.at[pl.ds(r0, rows_per_s)]
        )

        @pl.when(s == NS - 1)
        def _():
            pltpu.sync_copy(
                zero_hbm.at[pl.ds(tail_off, tail_rows)],
                acc_sh.at[pl.ds(tail_off, tail_rows)],
            )

        plsc.subcore_barrier()

        @pl.loop(0, n_ch)
        def _(i):
            off = s * per_s + i * CH
            pltpu.sync_copy(dst_hbm.at[pl.ds(off, CH)], idx_v)
            pltpu.sync_copy(pay_hbm.at[c, pl.ds(off, CH)], rows_v)
            pltpu.sync_copy(rows_v, acc_sh.at[idx_v], add=True)

        plsc.subcore_barrier()
        pltpu.sync_copy(
            acc_sh.at[pl.ds(r0, rows_per_s)], out_hbm.at[c, pl.ds(r0, rows_per_s)]
        )

        @pl.when(s == NS - 1)
        def _():
            pltpu.sync_copy(
                acc_sh.at[pl.ds(tail_off, tail_rows)],
                out_hbm.at[c, pl.ds(tail_off, tail_rows)],
            )

    return k(payload, dst, zeros)


def _tc_node_proj(x_s, w, b):
    n = x_s.shape[0]
    BN = 2000

    def body(x_ref, w_ref, b_ref, o_ref):
        o_ref[...] = (
            jnp.dot(x_ref[...], w_ref[...], preferred_element_type=jnp.float32)
            + b_ref[...]
        )

    return pl.pallas_call(
        body,
        grid=(n // BN,),
        in_specs=[
            pl.BlockSpec((BN, H), lambda i: (i, 0)),
            pl.BlockSpec((H, H), lambda i: (0, 0)),
            pl.BlockSpec((1, H), lambda i: (0, 0)),
        ],
        out_specs=pl.BlockSpec((BN, H), lambda i: (i, 0)),
        out_shape=jax.ShapeDtypeStruct((n, H), jnp.float32),
    )(x_s, w, b.reshape(1, H))


def _tc_edge(edge_attr, sphe_emb, torsion_emb, dist, v_g, p):
    E = edge_attr.shape[0]
    BE = 2000
    sphe_d = sphe_emb.shape[1]
    tors_d = torsion_emb.shape[1]
    mid = p["sphe_w1"].shape[1]

    def body(ea_ref, sp_ref, to_ref, di_ref, vg_ref,
             sw1_ref, sw2_ref, tw1_ref, tw2_ref,
             dvw_ref, dvb_ref, smw_ref, smb_ref,
             cma_ref, cmb_ref,
             fpw_ref, fpb_ref, cfa_ref, cfb_ref, cfbias_ref,
             fji_ref, pay_ref):
        dot = functools.partial(jnp.dot, preferred_element_type=jnp.float32)
        ea = ea_ref[...]
        d1 = dot(dot(sp_ref[...], sw1_ref[...]), sw2_ref[...])
        d2 = dot(dot(to_ref[...], tw1_ref[...]), tw2_ref[...])
        r = di_ref[...]
        cut = 0.5 * (jnp.cos(jnp.pi * r / CUT) + 1.0) * (r < CUT).astype(r.dtype)
        dv = _silu(dot(ea, dvw_ref[...]) + dvb_ref[...]) * cut
        v_j = vg_ref[...] * dv
        s = _silu(dot(v_j, smw_ref[...]) + smb_ref[...])
        pay_d = dot(s[:, :H] * d1, cma_ref[...]) + dot(s[:, H:] * d2, cmb_ref[...])
        f = _silu(dot(ea, fpw_ref[...]) + fpb_ref[...])
        f_ji = f[:, :H] + _silu(
            dot(f[:, H:2 * H] * d1, cfa_ref[...])
            + dot(f[:, 2 * H:] * d2, cfb_ref[...])
            + cfbias_ref[...]
        )
        fji_ref[...] = f_ji
        pay_ref[0] = v_j
        pay_ref[1] = pay_d

    def full(shape):
        return pl.BlockSpec(shape, lambda *_: tuple(0 for _ in shape))

    out = pl.pallas_call(
        body,
        grid=(E // BE,),
        in_specs=[
            pl.BlockSpec((BE, H), lambda i: (i, 0)),
            pl.BlockSpec((BE, sphe_d), lambda i: (i, 0)),
            pl.BlockSpec((BE, tors_d), lambda i: (i, 0)),
            pl.BlockSpec((BE, 1), lambda i: (i, 0)),
            pl.BlockSpec((BE, H), lambda i: (i, 0)),
            full((sphe_d, mid)), full((mid, H)),
            full((tors_d, mid)), full((mid, H)),
            full((H, H)), full((1, H)),
            full((H, 2 * H)), full((1, 2 * H)),
            full((H, H)), full((H, H)),
            full((H, 3 * H)), full((1, 3 * H)),
            full((H, H)), full((H, H)), full((1, H)),
        ],
        out_specs=[
            pl.BlockSpec((BE, H), lambda i: (i, 0)),
            pl.BlockSpec((2, BE, H), lambda i: (0, i, 0)),
        ],
        out_shape=[
            jax.ShapeDtypeStruct((E, H), jnp.float32),
            jax.ShapeDtypeStruct((2, E, H), jnp.float32),
        ],
    )(
        edge_attr, sphe_emb, torsion_emb, dist.reshape(E, 1), v_g,
        p["sphe_w1"], p["sphe_w2"], p["tors_w1"], p["tors_w2"],
        p["dv_w"], p["dv_b"].reshape(1, H),
        p["s_msg_w"], p["s_msg_b"].reshape(1, 2 * H),
        p["cat_msg_w"][:H], p["cat_msg_w"][H:],
        p["f_proj_w"], p["f_proj_b"].reshape(1, 3 * H),
        p["cat_f_w"][:H], p["cat_f_w"][H:], p["cat_f_b"].reshape(1, H),
    )
    return out[0], out[1]


def _tc_node_update(x_t, agg, p):
    n = x_t.shape[0]
    BN = 2000

    def body(xt_ref, agg_ref, omw_ref, omb_ref, fw_ref, fb_ref, o_ref):
        dot = functools.partial(jnp.dot, preferred_element_type=jnp.float32)
        s_msg = agg_ref[0]
        d_msg = agg_ref[1]
        om = dot(s_msg, omw_ref[...]) + omb_ref[...]
        pre = xt_ref[...] + om[:, :H] + om[:, H:] * d_msg
        o_ref[...] = _silu(dot(pre, fw_ref[...]) + fb_ref[...])

    return pl.pallas_call(
        body,
        grid=(n // BN,),
        in_specs=[
            pl.BlockSpec((BN, H), lambda i: (i, 0)),
            pl.BlockSpec((2, BN, H), lambda i: (0, i, 0)),
            pl.BlockSpec((H, 2 * H), lambda i: (0, 0)),
            pl.BlockSpec((1, 2 * H), lambda i: (0, 0)),
            pl.BlockSpec((H, H), lambda i: (0, 0)),
            pl.BlockSpec((1, H), lambda i: (0, 0)),
        ],
        out_specs=pl.BlockSpec((BN, H), lambda i: (i, 0)),
        out_shape=jax.ShapeDtypeStruct((n, H), jnp.float32),
    )(x_t, agg, p["o_msg_w"], p["o_msg_b"].reshape(1, 2 * H),
      p["final_w"], p["final_b"].reshape(1, H))


def kernel(x_s, x_t, edge_index, edge_attr, sphe_emb, torsion_emb, dist, params):
    n = x_s.shape[0]
    src = edge_index[0].astype(jnp.int32)
    dst = edge_index[1].astype(jnp.int32)
    v = _tc_node_proj(x_s, params["node_w"], params["node_b"])
    v_g = _sc_gather(v, src)
    f_ji, pay = _tc_edge(edge_attr, sphe_emb, torsion_emb, dist, v_g, params)
    agg = _sc_segsum(pay, dst, n)
    h_t = _tc_node_update(x_t, agg, params)
    return (h_t, f_ji)
